# baseline (speedup 1.0000x reference)
import jax
import jax.numpy as jnp
from jax import lax
from jax.experimental import pallas as pl
from jax.experimental.pallas import tpu as pltpu
from jax.experimental.pallas import tpu_sc as plsc

VOCAB = 1000
B = 1024
SEQ = 50
SEQ_PAD = 64
ROWS_PAD = 56
N_POS = B * SEQ
NC, NS, L = 2, 16, 16
NW = NC * NS
B_PER_W = B // NW
N_CHUNKS = B_PER_W


def _prep_body(table_ref, inp_ref, tgt_ref,
               lse_ref, tcopy_ref, fidxp_ref, idxp_ref):
    t = table_ref[...]
    m = jnp.max(t, axis=1)
    s = jnp.sum(jnp.exp(t - m[:, None]), axis=1)
    lse_ref[...] = m + jnp.log(s)
    tcopy_ref[...] = t
    inp = inp_ref[...]
    fidxp_ref[:, :SEQ] = inp * VOCAB + tgt_ref[...]
    fidxp_ref[:, SEQ:] = jnp.zeros((B, SEQ_PAD - SEQ), jnp.int32)
    idxp_ref[:, :SEQ] = inp
    idxp_ref[:, SEQ:] = jnp.zeros((B, SEQ_PAD - SEQ), jnp.int32)


def _finish_body(parts_ref, lse_ref, table_ref, loss_ref):
    corr = (B * (SEQ_PAD - SEQ)) * (lse_ref[0] - table_ref[0, 0])
    loss_ref[...] = (jnp.sum(parts_ref[...], keepdims=True) - corr) \
        * (1.0 / N_POS)


def _sc_body(table_hbm, tflat_hbm, fidxp_hbm, idxp_hbm, lse_hbm,
             out_hbm, part_hbm,
             fidxp_v, idxp_v, rows0, rows1, tv_v, lse_g_v, acc_v,
             gsem0, gsem1, wsem0, wsem1, lsem, lsem2):
    wid = lax.axis_index("s") * NC + lax.axis_index("c")
    b0 = wid * B_PER_W
    pltpu.sync_copy(fidxp_hbm.at[pl.ds(b0, B_PER_W)], fidxp_v)
    pltpu.sync_copy(idxp_hbm.at[pl.ds(b0, B_PER_W)], idxp_v)

    def tv_desc(g):
        return pltpu.make_async_copy(
            tflat_hbm.at[fidxp_v.at[g]], tv_v.at[pl.ds(g * SEQ_PAD, SEQ_PAD)],
            lsem)

    def ls_desc(g):
        return pltpu.make_async_copy(
            lse_hbm.at[idxp_v.at[g]], lse_g_v.at[pl.ds(g * SEQ_PAD, SEQ_PAD)],
            lsem2)

    tv_desc(0).start()
    ls_desc(0).start()
    tv_desc(1).start()
    ls_desc(1).start()

    bufs = (rows0, rows1)
    gsems = (gsem0, gsem1)
    wsems = (wsem0, wsem1)

    def g_desc(g, b):
        return pltpu.make_async_copy(
            table_hbm.at[idxp_v.at[g, pl.ds(0, ROWS_PAD)]],
            bufs[b], gsems[b])

    def w_desc(g, b):
        return pltpu.make_async_copy(
            bufs[b].at[pl.ds(0, SEQ)], out_hbm.at[b0 + g], wsems[b])

    g_desc(0, 0).start()
    g_desc(1, 1).start()
    g_desc(0, 0).wait()
    w_desc(0, 0).start()

    def pair(p, _):
        for k in (1, 2):
            g = 2 * p + k
            b = k % 2
            bp = 1 - b
            g_desc(g, b).wait()
            w_desc(g - 1, bp).wait()
            g_desc(g + 1, bp).start()
            w_desc(g, b).start()
            tv_desc(g + 1).start()
            ls_desc(g + 1).start()
            tv_desc(g - 1).wait()
            ls_desc(g - 1).wait()
        return 0

    lax.fori_loop(0, (N_CHUNKS - 2) // 2, pair, 0)

    last = N_CHUNKS - 1
    g_desc(last, 1).wait()
    w_desc(last - 1, 0).wait()
    w_desc(last, 1).start()
    w_desc(last, 1).wait()

    for g in (last - 1, last):
        tv_desc(g).wait()
        ls_desc(g).wait()
    def lbody(i, acc):
        sl = pl.ds(i * L, L)
        return acc + (lse_g_v[sl] - tv_v[sl])

    acc = lax.fori_loop(0, B_PER_W * SEQ_PAD // L, lbody,
                        jnp.zeros((L,), jnp.float32))
    acc_v[...] = acc
    pltpu.sync_copy(acc_v, part_hbm.at[wid])


def kernel(inputs, targets, table):
    lse, tcopy, fidxp, idxp = pl.pallas_call(
        _prep_body,
        out_shape=(
            jax.ShapeDtypeStruct((VOCAB,), jnp.float32),
            jax.ShapeDtypeStruct((VOCAB, VOCAB), jnp.float32),
            jax.ShapeDtypeStruct((B, SEQ_PAD), jnp.int32),
            jax.ShapeDtypeStruct((B, SEQ_PAD), jnp.int32),
        ),
    )(table, inputs, targets)
    table_flat = tcopy.reshape(-1)

    mesh = plsc.VectorSubcoreMesh(core_axis_name="c", subcore_axis_name="s")
    sc = pl.kernel(
        _sc_body,
        out_type=(
            jax.ShapeDtypeStruct((B, SEQ, VOCAB), jnp.float32),
            jax.ShapeDtypeStruct((NW, L), jnp.float32),
        ),
        mesh=mesh,
        compiler_params=pltpu.CompilerParams(use_tc_tiling_on_sc=False),
        scratch_types=[
            pltpu.VMEM((B_PER_W, SEQ_PAD), jnp.int32),
            pltpu.VMEM((B_PER_W, SEQ_PAD), jnp.int32),
            pltpu.VMEM((ROWS_PAD, VOCAB), jnp.float32),
            pltpu.VMEM((ROWS_PAD, VOCAB), jnp.float32),
            pltpu.VMEM((B_PER_W * SEQ_PAD,), jnp.float32),
            pltpu.VMEM((B_PER_W * SEQ_PAD,), jnp.float32),
            pltpu.VMEM((L,), jnp.float32),
            pltpu.SemaphoreType.DMA,
            pltpu.SemaphoreType.DMA,
            pltpu.SemaphoreType.DMA,
            pltpu.SemaphoreType.DMA,
            pltpu.SemaphoreType.DMA,
            pltpu.SemaphoreType.DMA,
        ],
    )
    logits, parts = sc(table, table_flat, fidxp, idxp, lse)

    loss = pl.pallas_call(
        _finish_body,
        out_shape=jax.ShapeDtypeStruct((1, 1), jnp.float32),
    )(parts, lse, table)[0, 0]

    return logits, loss


# speedup vs baseline: 1.3941x; 1.3941x over previous
import jax
import jax.numpy as jnp
from jax import lax
from jax.experimental import pallas as pl
from jax.experimental.pallas import tpu as pltpu
from jax.experimental.pallas import tpu_sc as plsc

VOCAB = 1000
VOCAB_PAD = 1024
B = 1024
SEQ = 50
SEQ_PAD = 64
ROWS_PAD = 56
N_POS = B * SEQ
NC, NS, L = 2, 16, 16
NW = NC * NS
B_PER_W = B // NW
N_CHUNKS = B_PER_W


def _prep_body(table_ref, inp_ref, tgt_ref,
               lse_ref, tpad_ref, fidxp_ref, idxp_ref):
    t = table_ref[...]
    m = jnp.max(t, axis=1)
    s = jnp.sum(jnp.exp(t - m[:, None]), axis=1)
    lse_ref[...] = m + jnp.log(s)
    tpad_ref[:, :VOCAB] = t
    tpad_ref[:, VOCAB:] = jnp.zeros((VOCAB, VOCAB_PAD - VOCAB), jnp.float32)
    inp = inp_ref[...]
    fidxp_ref[:, :SEQ] = inp * VOCAB_PAD + tgt_ref[...]
    fidxp_ref[:, SEQ:] = jnp.zeros((B, SEQ_PAD - SEQ), jnp.int32)
    idxp_ref[:, :SEQ] = inp
    idxp_ref[:, SEQ:] = jnp.zeros((B, SEQ_PAD - SEQ), jnp.int32)


def _finish_body(parts_ref, lse_ref, table_ref, loss_ref):
    corr = (B * (SEQ_PAD - SEQ)) * (lse_ref[0] - table_ref[0, 0])
    val = (jnp.sum(parts_ref[...]) - corr) * (1.0 / N_POS)
    loss_ref[...] = val * jnp.ones((1, 1), jnp.float32)


def _sc_body(tpad_hbm, tflat_hbm, fidxp_hbm, idxp_hbm, lse_hbm,
             out_hbm, part_hbm,
             fidxp_v, idxp_v, rows0, rows1, tv_v, lse_g_v, acc_v,
             gsem0, gsem1, wsem0, wsem1, lsem, lsem2):
    wid = lax.axis_index("s") * NC + lax.axis_index("c")
    b0 = wid * B_PER_W
    pltpu.sync_copy(fidxp_hbm.at[pl.ds(b0, B_PER_W)], fidxp_v)
    pltpu.sync_copy(idxp_hbm.at[pl.ds(b0, B_PER_W)], idxp_v)

    def tv_desc(g):
        return pltpu.make_async_copy(
            tflat_hbm.at[fidxp_v.at[g]], tv_v.at[pl.ds(g * SEQ_PAD, SEQ_PAD)],
            lsem)

    def ls_desc(g):
        return pltpu.make_async_copy(
            lse_hbm.at[idxp_v.at[g]], lse_g_v.at[pl.ds(g * SEQ_PAD, SEQ_PAD)],
            lsem2)

    tv_desc(0).start()
    ls_desc(0).start()
    tv_desc(1).start()
    ls_desc(1).start()

    bufs = (rows0, rows1)
    gsems = (gsem0, gsem1)
    wsems = (wsem0, wsem1)

    def g_desc(g, b):
        return pltpu.make_async_copy(
            tpad_hbm.at[idxp_v.at[g, pl.ds(0, ROWS_PAD)]],
            bufs[b], gsems[b])

    def w_desc(g, b):
        return pltpu.make_async_copy(
            bufs[b], out_hbm.at[b0 + g], wsems[b])

    g_desc(0, 0).start()
    g_desc(1, 1).start()
    g_desc(0, 0).wait()
    w_desc(0, 0).start()

    def pair(p, _):
        for k in (1, 2):
            g = 2 * p + k
            b = k % 2
            bp = 1 - b
            g_desc(g, b).wait()
            w_desc(g - 1, bp).wait()
            g_desc(g + 1, bp).start()
            w_desc(g, b).start()
            tv_desc(g + 1).start()
            ls_desc(g + 1).start()
            tv_desc(g - 1).wait()
            ls_desc(g - 1).wait()
        return 0

    lax.fori_loop(0, (N_CHUNKS - 2) // 2, pair, 0)

    last = N_CHUNKS - 1
    g_desc(last, 1).wait()
    w_desc(last - 1, 0).wait()
    w_desc(last, 1).start()
    w_desc(last, 1).wait()

    for g in (last - 1, last):
        tv_desc(g).wait()
        ls_desc(g).wait()

    def lbody(i, acc):
        sl = pl.ds(i * L, L)
        return acc + (lse_g_v[sl] - tv_v[sl])

    acc = lax.fori_loop(0, B_PER_W * SEQ_PAD // L, lbody,
                        jnp.zeros((L,), jnp.float32))
    acc_v[...] = acc
    pltpu.sync_copy(acc_v, part_hbm.at[pl.ds(wid * L, L)])


def kernel(inputs, targets, table):
    lse, tpad, fidxp, idxp = pl.pallas_call(
        _prep_body,
        out_shape=(
            jax.ShapeDtypeStruct((VOCAB,), jnp.float32),
            jax.ShapeDtypeStruct((VOCAB, VOCAB_PAD), jnp.float32),
            jax.ShapeDtypeStruct((B, SEQ_PAD), jnp.int32),
            jax.ShapeDtypeStruct((B, SEQ_PAD), jnp.int32),
        ),
    )(table, inputs, targets)
    table_flat = tpad.reshape(-1)

    mesh = plsc.VectorSubcoreMesh(core_axis_name="c", subcore_axis_name="s")
    sc = pl.kernel(
        _sc_body,
        out_type=(
            jax.ShapeDtypeStruct((B, ROWS_PAD, VOCAB_PAD), jnp.float32),
            jax.ShapeDtypeStruct((NW * L,), jnp.float32),
        ),
        mesh=mesh,
        compiler_params=pltpu.CompilerParams(use_tc_tiling_on_sc=True),
        scratch_types=[
            pltpu.VMEM((B_PER_W, SEQ_PAD), jnp.int32),
            pltpu.VMEM((B_PER_W, SEQ_PAD), jnp.int32),
            pltpu.VMEM((ROWS_PAD, VOCAB_PAD), jnp.float32),
            pltpu.VMEM((ROWS_PAD, VOCAB_PAD), jnp.float32),
            pltpu.VMEM((B_PER_W * SEQ_PAD,), jnp.float32),
            pltpu.VMEM((B_PER_W * SEQ_PAD,), jnp.float32),
            pltpu.VMEM((L,), jnp.float32),
            pltpu.SemaphoreType.DMA,
            pltpu.SemaphoreType.DMA,
            pltpu.SemaphoreType.DMA,
            pltpu.SemaphoreType.DMA,
            pltpu.SemaphoreType.DMA,
            pltpu.SemaphoreType.DMA,
        ],
    )
    logits_pad, parts = sc(tpad, table_flat, fidxp, idxp, lse)

    loss = pl.pallas_call(
        _finish_body,
        out_shape=jax.ShapeDtypeStruct((1, 1), jnp.float32),
    )(parts, lse, table)[0, 0]

    return logits_pad[:, :SEQ, :VOCAB], loss
